# baseline (speedup 1.0000x reference)
import jax
import jax.numpy as jnp
from jax.experimental import pallas as pl

_GAMMA = 2.0
_ITERS = 24
_ROWS_PER_BLOCK = 64
_STRIP_WIDTH = 512


def _sparsemax_block(x_ref, o_ref):
    r, dim = x_ref.shape
    w = _STRIP_WIDTH

    m_acc = jnp.full((r, w), -jnp.inf, jnp.float32)
    for j in range(dim // w):
        m_acc = jnp.maximum(m_acc, x_ref[:, j * w:(j + 1) * w])
    rowmax = jnp.max(m_acc, axis=1, keepdims=True)
    t0 = rowmax - (1.0 / _GAMMA)

    def step(t):
        s_acc = jnp.zeros((r, w), jnp.float32)
        c_acc = jnp.zeros((r, w), jnp.float32)
        for j in range(dim // w):
            xc = x_ref[:, j * w:(j + 1) * w]
            m = xc > t
            s_acc = s_acc + jnp.where(m, xc, 0.0)
            c_acc = c_acc + jnp.where(m, 1.0, 0.0)
        s = jnp.sum(s_acc, axis=1, keepdims=True)
        c = jnp.sum(c_acc, axis=1, keepdims=True)
        return (s - (1.0 / _GAMMA)) / c

    def cond(carry):
        k, t_prev, t = carry
        return jnp.logical_and(k < _ITERS, jnp.any(t_prev != t))

    def body(carry):
        k, _, t = carry
        return k + 1, t, step(t)

    _, _, t = jax.lax.while_loop(cond, body, (0, t0 - 1.0, t0))
    w2 = 2 * w
    for j in range(dim // w2):
        sl = slice(j * w2, (j + 1) * w2)
        o_ref[:, sl] = _GAMMA * jnp.maximum(x_ref[:, sl] - t, 0.0)


def kernel(input):
    bs, dim = input.shape
    r = _ROWS_PER_BLOCK
    return pl.pallas_call(
        _sparsemax_block,
        grid=(bs // r,),
        in_specs=[pl.BlockSpec((r, dim), lambda i: (i, 0))],
        out_specs=pl.BlockSpec((r, dim), lambda i: (i, 0)),
        out_shape=jax.ShapeDtypeStruct((bs, dim), input.dtype),
    )(input)


# speedup vs baseline: 1.0385x; 1.0385x over previous
import jax
import jax.numpy as jnp
from jax.experimental import pallas as pl

_GAMMA = 2.0
_ITERS = 24
_ROWS_PER_BLOCK = 32
_STRIP_WIDTH = 512


def _sparsemax_block(x_ref, o_ref):
    r, dim = x_ref.shape
    w = _STRIP_WIDTH

    m_acc = jnp.full((r, w), -jnp.inf, jnp.float32)
    for j in range(dim // w):
        m_acc = jnp.maximum(m_acc, x_ref[:, j * w:(j + 1) * w])
    rowmax = jnp.max(m_acc, axis=1, keepdims=True)
    t0 = rowmax - (1.0 / _GAMMA)

    def step(t):
        s_acc = jnp.zeros((r, w), jnp.float32)
        c_acc = jnp.zeros((r, w), jnp.float32)
        for j in range(dim // w):
            xc = x_ref[:, j * w:(j + 1) * w]
            m = xc > t
            s_acc = s_acc + jnp.where(m, xc, 0.0)
            c_acc = c_acc + jnp.where(m, 1.0, 0.0)
        s = jnp.sum(s_acc, axis=1, keepdims=True)
        c = jnp.sum(c_acc, axis=1, keepdims=True)
        return (s - (1.0 / _GAMMA)) / c

    def cond(carry):
        k, t_prev, t = carry
        return jnp.logical_and(k < _ITERS, jnp.any(t_prev != t))

    def body(carry):
        k, _, t = carry
        return k + 1, t, step(t)

    _, _, t = jax.lax.while_loop(cond, body, (0, t0 - 1.0, t0))
    w2 = 2 * w
    for j in range(dim // w2):
        sl = slice(j * w2, (j + 1) * w2)
        o_ref[:, sl] = _GAMMA * jnp.maximum(x_ref[:, sl] - t, 0.0)


def kernel(input):
    bs, dim = input.shape
    r = _ROWS_PER_BLOCK
    return pl.pallas_call(
        _sparsemax_block,
        grid=(bs // r,),
        in_specs=[pl.BlockSpec((r, dim), lambda i: (i, 0))],
        out_specs=pl.BlockSpec((r, dim), lambda i: (i, 0)),
        out_shape=jax.ShapeDtypeStruct((bs, dim), input.dtype),
    )(input)
